# baseline (speedup 1.0000x reference)
import functools

import jax
import jax.numpy as jnp
from jax import lax
from jax.experimental import pallas as pl
from jax.experimental.pallas import tpu as pltpu
from jax.experimental.pallas import tpu_sc as plsc

_D = 64
_CH = 160
_NB = 5
_AH = 3

_info = plsc.get_sparse_core_info()
_NC = _info.num_cores
_NS = _info.num_subcores
_NW = _NC * _NS


def _make_lookup(n_rows):
    n_per_w = n_rows // _NW
    n_chunks = n_per_w // _CH
    mesh = plsc.VectorSubcoreMesh(core_axis_name="c", subcore_axis_name="s")

    @functools.partial(
        pl.kernel,
        mesh=mesh,
        out_type=jax.ShapeDtypeStruct((n_rows, _D), jnp.float32),
        scratch_types=(
            [pltpu.VMEM((_CH,), jnp.int32) for _ in range(_NB)]
            + [pltpu.VMEM((_CH, _D), jnp.float32) for _ in range(_NB)]
            + [pltpu.SemaphoreType.DMA for _ in range(2 * _NB)]
        ),
    )
    def lookup(idx_hbm, table_hbm, out_hbm, *scr):
        rv = scr[:_NB]
        gb = scr[_NB:2 * _NB]
        gsem = scr[2 * _NB:3 * _NB]
        wsem = scr[3 * _NB:]
        wid = lax.axis_index("s") * _NC + lax.axis_index("c")
        base = pl.multiple_of(wid * n_per_w, n_per_w)

        def prep(c, b):
            pltpu.sync_copy(idx_hbm.at[pl.ds(base + c * _CH, _CH)], rv[b])
            for g in range(_CH // 16):
                tv = rv[b][pl.ds(g * 16, 16)]
                hi = lax.shift_right_logical(tv, 3)
                lo = lax.bitwise_and(tv, 7)
                for u in range(16):
                    pltpu.async_copy(
                        table_hbm.at[hi[u], lo[u]],
                        gb[b].at[g * 16 + u],
                        gsem[b],
                    )

        for p in range(_AH):
            prep(p, p)

        def outer(cg, _):
            for b in range(_NB):
                c = cg * _NB + b
                nxt = (b + _AH) % _NB

                @pl.when(c + _AH < n_chunks)
                def _():
                    @pl.when(c + _AH >= _NB)
                    def _():
                        pltpu.make_async_copy(
                            gb[nxt], out_hbm.at[pl.ds(0, _CH)], wsem[nxt]
                        ).wait()

                    prep(c + _AH, nxt)

                pltpu.make_async_copy(
                    out_hbm.at[pl.ds(0, _CH)], gb[b], gsem[b]
                ).wait()

                pltpu.async_copy(
                    gb[b], out_hbm.at[pl.ds(base + c * _CH, _CH)], wsem[b]
                )
            return ()

        lax.fori_loop(0, n_chunks // _NB, outer, ())
        for c in range(n_chunks - _NB, n_chunks):
            pltpu.make_async_copy(
                gb[c % _NB], out_hbm.at[pl.ds(0, _CH)], wsem[c % _NB]
            ).wait()

    return lookup


def kernel(token_ids, weight):
    n_rows = token_ids.size
    idx = token_ids.reshape(n_rows)
    table = weight.reshape(weight.shape[0] // 8, 8, weight.shape[1])
    out = _make_lookup(n_rows)(idx, table)
    return out.reshape(token_ids.shape + (weight.shape[1],))


# speedup vs baseline: 1.0354x; 1.0354x over previous
import functools

import jax
import jax.numpy as jnp
from jax import lax
from jax.experimental import pallas as pl
from jax.experimental.pallas import tpu as pltpu
from jax.experimental.pallas import tpu_sc as plsc

_D = 64
_CH = 160
_NB = 4

_info = plsc.get_sparse_core_info()
_NC = _info.num_cores
_NS = _info.num_subcores
_NW = _NC * _NS


def _make_lookup(n_rows):
    n_per_w = n_rows // _NW
    n_chunks = n_per_w // _CH
    mesh = plsc.VectorSubcoreMesh(core_axis_name="c", subcore_axis_name="s")

    @functools.partial(
        pl.kernel,
        mesh=mesh,
        out_type=jax.ShapeDtypeStruct((n_rows, _D), jnp.float32),
        scratch_types=(
            [pltpu.VMEM((_CH,), jnp.int32) for _ in range(_NB)]
            + [pltpu.VMEM((_CH, _D), jnp.float32) for _ in range(_NB)]
            + [pltpu.SemaphoreType.DMA for _ in range(2 * _NB)]
        ),
    )
    def lookup(idx_hbm, table_hbm, out_hbm, *scr):
        rv = scr[:_NB]
        gb = scr[_NB:2 * _NB]
        gsem = scr[2 * _NB:3 * _NB]
        wsem = scr[3 * _NB:]
        wid = lax.axis_index("s") * _NC + lax.axis_index("c")
        base = pl.multiple_of(wid * n_per_w, n_per_w)

        def prep(c, b):
            pltpu.sync_copy(idx_hbm.at[pl.ds(base + c * _CH, _CH)], rv[b])
            for g in range(_CH // 16):
                tv = rv[b][pl.ds(g * 16, 16)]
                hi = lax.shift_right_logical(tv, 3)
                lo = lax.bitwise_and(tv, 7)
                for u in range(16):
                    pltpu.async_copy(
                        table_hbm.at[hi[u], lo[u]],
                        gb[b].at[g * 16 + u],
                        gsem[b],
                    )

        prep(0, 0)
        prep(1, 1)

        def outer(cg, _):
            for b in range(_NB):
                c = cg * _NB + b
                nxt = (b + 2) % _NB

                @pl.when(c + 2 < n_chunks)
                def _():
                    @pl.when(c >= 2)
                    def _():
                        pltpu.make_async_copy(
                            gb[nxt], out_hbm.at[pl.ds(0, _CH)], wsem[nxt]
                        ).wait()

                    prep(c + 2, nxt)

                pltpu.make_async_copy(
                    out_hbm.at[pl.ds(0, _CH)], gb[b], gsem[b]
                ).wait()

                pltpu.async_copy(
                    gb[b], out_hbm.at[pl.ds(base + c * _CH, _CH)], wsem[b]
                )
            return ()

        lax.fori_loop(0, n_chunks // _NB, outer, ())
        for c in range(n_chunks - 4, n_chunks):
            pltpu.make_async_copy(
                gb[c % _NB], out_hbm.at[pl.ds(0, _CH)], wsem[c % _NB]
            ).wait()

    return lookup


def kernel(token_ids, weight):
    n_rows = token_ids.size
    idx = token_ids.reshape(n_rows)
    table = weight.reshape(weight.shape[0] // 8, 8, weight.shape[1])
    out = _make_lookup(n_rows)(idx, table)
    return out.reshape(token_ids.shape + (weight.shape[1],))
